# Initial kernel scaffold; baseline (speedup 1.0000x reference)
import jax
import jax.numpy as jnp
from jax.experimental import pallas as pl


def kernel(x, table):
    raise NotImplementedError("write your pallas kernel here")


# speedup vs baseline: 1.0758x; 1.0758x over previous
import functools

import jax
import jax.numpy as jnp
from jax import lax
from jax.experimental import pallas as pl
from jax.experimental.pallas import tpu as pltpu
from jax.experimental.pallas import tpu_sc as plsc

D = 100
DP = 128
CHUNK = 128

_info = plsc.get_sparse_core_info()
_NC, _NS = _info.num_cores, _info.num_subcores
NW = _NC * _NS


def _emb_call(n_total):
    n_per_w = n_total // NW
    n_chunks = n_per_w // CHUNK
    assert n_chunks % 2 == 0
    mesh = plsc.VectorSubcoreMesh(core_axis_name="c", subcore_axis_name="s")

    @functools.partial(
        pl.kernel,
        out_type=jax.ShapeDtypeStruct((n_total, DP), jnp.float32),
        mesh=mesh,
        scratch_types=[
            pltpu.VMEM((n_chunks, CHUNK), jnp.int32),
            pltpu.VMEM((2, CHUNK, DP), jnp.float32),
            pltpu.SemaphoreType.DMA,
            pltpu.SemaphoreType.DMA,
            pltpu.SemaphoreType.DMA,
            pltpu.SemaphoreType.DMA,
        ],
        compiler_params=pltpu.CompilerParams(use_tc_tiling_on_sc=False),
    )
    def emb(idx_hbm, table_hbm, out_hbm, idx_v, rows_v, g0, g1, o0, o1):
        wid = lax.axis_index("s") * _NC + lax.axis_index("c")
        base = wid * n_per_w
        gsems = (g0, g1)
        osems = (o0, o1)
        pltpu.sync_copy(idx_hbm.at[wid], idx_v)

        def gather(j, slot):
            return pltpu.make_async_copy(
                table_hbm.at[idx_v.at[j]], rows_v.at[slot], gsems[slot])

        def put(j, slot):
            return pltpu.make_async_copy(
                rows_v.at[slot], out_hbm.at[pl.ds(base + j * CHUNK, CHUNK)],
                osems[slot])

        gather(0, 0).start()
        gather(1, 1).start()

        def body(i, carry):
            j0 = i * 2
            for slot in range(2):
                gather(j0 + slot, slot).wait()
                put(j0 + slot, slot).start()
            for slot in range(2):
                put(j0 + slot, slot).wait()
                gather(j0 + 2 + slot, slot).start()
            return carry

        lax.fori_loop(0, n_chunks // 2 - 1, body, 0)

        jlast = n_chunks - 2
        for slot in range(2):
            gather(jlast + slot, slot).wait()
            put(jlast + slot, slot).start()
        for slot in range(2):
            put(jlast + slot, slot).wait()

    return emb


def kernel(x, table):
    b, s = x.shape
    n_total = b * s
    idx = x.reshape(NW, n_total // NW // CHUNK, CHUNK).astype(jnp.int32)
    table_p = jnp.pad(table, ((0, 0), (0, DP - D)))
    out = _emb_call(n_total)(idx, table_p)
    return out[:, :D].reshape(b, s, D)


# speedup vs baseline: 1.1126x; 1.0343x over previous
import functools

import jax
import jax.numpy as jnp
from jax import lax
from jax.experimental import pallas as pl
from jax.experimental.pallas import tpu as pltpu
from jax.experimental.pallas import tpu_sc as plsc

D = 100
DP = 128
CHUNK = 128

_info = plsc.get_sparse_core_info()
_NC, _NS = _info.num_cores, _info.num_subcores
NW = _NC * _NS


def _emb_call(n_total):
    n_per_w = n_total // NW
    n_chunks = n_per_w // CHUNK
    NBUF = 4
    assert n_chunks % NBUF == 0
    mesh = plsc.VectorSubcoreMesh(core_axis_name="c", subcore_axis_name="s")

    @functools.partial(
        pl.kernel,
        out_type=jax.ShapeDtypeStruct((n_total, DP), jnp.float32),
        mesh=mesh,
        scratch_types=[
            pltpu.VMEM((n_chunks, CHUNK), jnp.int32),
            pltpu.VMEM((NBUF, CHUNK, DP), jnp.float32),
        ] + [pltpu.SemaphoreType.DMA] * (2 * NBUF),
        compiler_params=pltpu.CompilerParams(use_tc_tiling_on_sc=False),
    )
    def emb(idx_hbm, table_hbm, out_hbm, idx_v, rows_v, *sems):
        wid = lax.axis_index("s") * _NC + lax.axis_index("c")
        base = wid * n_per_w
        gsems = sems[:NBUF]
        osems = sems[NBUF:]
        pltpu.sync_copy(idx_hbm.at[wid], idx_v)

        def gather(j, slot):
            return pltpu.make_async_copy(
                table_hbm.at[idx_v.at[j]], rows_v.at[slot], gsems[slot])

        def put(j, slot):
            return pltpu.make_async_copy(
                rows_v.at[slot], out_hbm.at[pl.ds(base + j * CHUNK, CHUNK)],
                osems[slot])

        for slot in range(NBUF):
            gather(slot, slot).start()

        def body(i, carry):
            j0 = i * NBUF
            for slot in range(NBUF):
                gather(j0 + slot, slot).wait()
                put(j0 + slot, slot).start()
            for slot in range(NBUF):
                put(j0 + slot, slot).wait()
                gather(j0 + NBUF + slot, slot).start()
            return carry

        lax.fori_loop(0, n_chunks // NBUF - 1, body, 0)

        jlast = n_chunks - NBUF
        for slot in range(NBUF):
            gather(jlast + slot, slot).wait()
            put(jlast + slot, slot).start()
        for slot in range(NBUF):
            put(jlast + slot, slot).wait()

    return emb


def kernel(x, table):
    b, s = x.shape
    n_total = b * s
    idx = x.reshape(NW, n_total // NW // CHUNK, CHUNK).astype(jnp.int32)
    table_p = jnp.pad(table, ((0, 0), (0, DP - D)))
    out = _emb_call(n_total)(idx, table_p)
    return out[:, :D].reshape(b, s, D)
